# baseline (speedup 1.0000x reference)
import functools

import jax
import jax.numpy as jnp
from jax import lax
from jax.experimental import pallas as pl
from jax.experimental.pallas import tpu as pltpu
from jax.experimental.pallas import tpu_sc as plsc

E = 8
K = 2
D = 768
H = 768
S = 2048
NPAIR = S * K
TMC = 128
NP = NPAIR + E * TMC
NT = NPAIR // TMC + E
NC = 2
NS = 16
NW = NC * NS
BP = NPAIR // NW
TW = S // NW
SUB = 32


def _router_kernel(x_ref, wr1_ref, br1_ref, wg_ref,
                   dpos_ref, gp_ref, cnt_ref):
    xb = x_ref[...]
    h = lax.dot_general(xb, wr1_ref[...], (((1,), (1,)), ((), ())),
                        preferred_element_type=jnp.float32)
    h = jnp.tanh(h + br1_ref[...])
    logits = lax.dot_general(h, wg_ref[...], (((1,), (1,)), ((), ())),
                             preferred_element_type=jnp.float32)
    m = jnp.max(logits, axis=1, keepdims=True)
    p = jnp.exp(logits - m)
    p = p / jnp.sum(p, axis=1, keepdims=True)
    e_iota = lax.broadcasted_iota(jnp.int32, p.shape, 1)
    m1 = jnp.max(p, axis=1, keepdims=True)
    i1 = jnp.min(jnp.where(p == m1, e_iota, E), axis=1, keepdims=True)
    p_rest = jnp.where(e_iota == i1, -jnp.inf, p)
    m2 = jnp.max(p_rest, axis=1, keepdims=True)
    i2 = jnp.min(jnp.where(p_rest == m2, e_iota, E), axis=1, keepdims=True)
    denom = m1 + m2 + 1e-6
    g1 = m1 / denom
    g2 = m2 / denom

    e8 = lax.broadcasted_iota(jnp.int32, (S, E), 1)
    oh1 = (e8 == i1).astype(jnp.float32)
    oh2 = (e8 == i2).astype(jnp.float32)
    oh = jnp.concatenate([oh1, oh2], axis=0)

    r_i = lax.broadcasted_iota(jnp.int32, (256, 256), 0)
    c_i = lax.broadcasted_iota(jnp.int32, (256, 256), 1)
    tri = (c_i < r_i).astype(jnp.float32)

    nblk = NPAIR // 256
    t_rows = []
    rank_blocks = []
    for b in range(nblk):
        ohb = oh[b * 256:(b + 1) * 256, :]
        rb = lax.dot_general(tri, ohb, (((1,), (0,)), ((), ())),
                             preferred_element_type=jnp.float32)
        rank_blocks.append(rb)
        t_rows.append(jnp.sum(ohb, axis=0, keepdims=True))
    t_mat = jnp.concatenate(t_rows, axis=0)

    r16 = lax.broadcasted_iota(jnp.int32, (nblk, nblk), 0)
    c16 = lax.broadcasted_iota(jnp.int32, (nblk, nblk), 1)
    tri16 = (r16 < c16).astype(jnp.float32)
    bo = lax.dot_general(tri16, t_mat, (((0,), (0,)), ((), ())),
                         preferred_element_type=jnp.float32)

    counts = jnp.sum(t_mat, axis=0, keepdims=True)
    ci = counts.astype(jnp.int32)
    pci = ((ci + (TMC - 1)) // TMC) * TMC
    pcf = pci.astype(jnp.float32)
    r8 = lax.broadcasted_iota(jnp.int32, (E, E), 0)
    c8 = lax.broadcasted_iota(jnp.int32, (E, E), 1)
    tri8 = (r8 < c8).astype(jnp.float32)
    eo = lax.dot_general(pcf, tri8, (((1,), (0,)), ((), ())),
                         preferred_element_type=jnp.float32)

    dpos_parts = []
    for b in range(nblk):
        ohb = oh[b * 256:(b + 1) * 256, :]
        base = rank_blocks[b] + bo[b:b + 1, :] + eo
        dpos_parts.append(jnp.sum(base * ohb, axis=1, keepdims=True))
    dpos = jnp.concatenate(dpos_parts, axis=0)
    dpos_ref[...] = dpos.astype(jnp.int32)

    gpair = jnp.concatenate([g1, g2], axis=0)
    gp_ref[...] = jnp.broadcast_to(gpair, (NPAIR, 128))
    cnt_ref[...] = ci


def _router_call(x2d, Wr1, br1_2d, Wg, interpret=False):
    return pl.pallas_call(
        _router_kernel,
        in_specs=[
            pl.BlockSpec((S, D), lambda: (0, 0)),
            pl.BlockSpec((D, D), lambda: (0, 0)),
            pl.BlockSpec((1, D), lambda: (0, 0)),
            pl.BlockSpec((E, D), lambda: (0, 0)),
        ],
        out_specs=[
            pl.BlockSpec((NPAIR, 1), lambda: (0, 0)),
            pl.BlockSpec((NPAIR, 128), lambda: (0, 0)),
            pl.BlockSpec((1, E), lambda: (0, 0)),
        ],
        out_shape=[
            jax.ShapeDtypeStruct((NPAIR, 1), jnp.int32),
            jax.ShapeDtypeStruct((NPAIR, 128), jnp.float32),
            jax.ShapeDtypeStruct((1, E), jnp.int32),
        ],
        compiler_params=pltpu.CompilerParams(
            vmem_limit_bytes=60 * 1024 * 1024,
        ),
        interpret=interpret,
    )(x2d, Wr1, br1_2d, Wg)


def _scatter_kernel(dpos_hbm, x_hbm, gp_hbm, xs_hbm, gs_hbm,
                    idx_v, rows_v, g_v, sem1, sem2):
    wid = lax.axis_index("s") * NC + lax.axis_index("c")
    pbase = wid * BP
    tokbase = lax.rem(pbase, S)
    pltpu.sync_copy(dpos_hbm.at[pl.ds(pbase, BP)], idx_v)
    pltpu.sync_copy(x_hbm.at[pl.ds(tokbase, BP)], rows_v)
    pltpu.sync_copy(gp_hbm.at[pl.ds(pbase, BP)], g_v)
    cp1 = pltpu.async_copy(rows_v, xs_hbm.at[idx_v], sem1)
    cp2 = pltpu.async_copy(g_v, gs_hbm.at[idx_v], sem2)
    cp1.wait()
    cp2.wait()


@functools.lru_cache(maxsize=1)
def _scatter_call_builder():
    return pl.kernel(
        _scatter_kernel,
        mesh=plsc.VectorSubcoreMesh(core_axis_name="c", subcore_axis_name="s"),
        out_type=[
            jax.ShapeDtypeStruct((NP, D), jnp.float32),
            jax.ShapeDtypeStruct((NP, 128), jnp.float32),
        ],
        scratch_types=[
            pltpu.VMEM((BP,), jnp.int32),
            pltpu.VMEM((BP, D), jnp.float32),
            pltpu.VMEM((BP, 128), jnp.float32),
            pltpu.SemaphoreType.DMA,
            pltpu.SemaphoreType.DMA,
        ],
    )


def _scatter_call(dpos_hbm, x_hbm, gp_hbm):
    return _scatter_call_builder()(dpos_hbm, x_hbm, gp_hbm)


def _gmm_kernel(te_ref, xs_ref, gs_ref, win_ref, wout_ref, out_ref):
    del te_ref
    hh = lax.dot_general(xs_ref[...], win_ref[0], (((1,), (0,)), ((), ())),
                         preferred_element_type=jnp.float32)
    hh = jnp.maximum(hh, 0.0)
    oo = lax.dot_general(hh, wout_ref[0], (((1,), (0,)), ((), ())),
                         preferred_element_type=jnp.float32)
    out_ref[...] = oo * gs_ref[:, 0:1]


def _gmm_call(te, xs, gs, We_in, We_out, interpret=False):
    return pl.pallas_call(
        _gmm_kernel,
        grid_spec=pltpu.PrefetchScalarGridSpec(
            num_scalar_prefetch=1,
            grid=(NT,),
            in_specs=[
                pl.BlockSpec((TMC, D), lambda t, te_ref: (t, 0)),
                pl.BlockSpec((TMC, 128), lambda t, te_ref: (t, 0)),
                pl.BlockSpec((1, D, H), lambda t, te_ref: (te_ref[t], 0, 0)),
                pl.BlockSpec((1, H, D), lambda t, te_ref: (te_ref[t], 0, 0)),
            ],
            out_specs=pl.BlockSpec((TMC, D), lambda t, te_ref: (t, 0)),
        ),
        out_shape=jax.ShapeDtypeStruct((NP, D), jnp.float32),
        compiler_params=pltpu.CompilerParams(
            dimension_semantics=("arbitrary",),
            vmem_limit_bytes=60 * 1024 * 1024,
        ),
        interpret=interpret,
    )(te, xs, gs, We_in, We_out)


def _combine_kernel(dp0_hbm, dp1_hbm, outs_hbm, y_hbm,
                    p0_v, p1_v, r0_v, r1_v, sem0, sem1):
    wid = lax.axis_index("s") * NC + lax.axis_index("c")
    for sub in range(TW // SUB):
        base = wid * TW + sub * SUB
        pltpu.sync_copy(dp0_hbm.at[pl.ds(base, SUB)], p0_v)
        pltpu.sync_copy(dp1_hbm.at[pl.ds(base, SUB)], p1_v)
        cp0 = pltpu.async_copy(outs_hbm.at[p0_v], r0_v, sem0)
        cp1 = pltpu.async_copy(outs_hbm.at[p1_v], r1_v, sem1)
        cp0.wait()
        cp1.wait()

        def _row(r, carry):
            for j in range(D // 16):
                sl = pl.ds(j * 16, 16)
                r0_v[r, sl] = r0_v[r, sl] + r1_v[r, sl]
            return carry

        lax.fori_loop(0, SUB, _row, 0)
        pltpu.sync_copy(r0_v, y_hbm.at[pl.ds(base, SUB)])


@functools.lru_cache(maxsize=1)
def _combine_call_builder():
    return pl.kernel(
        _combine_kernel,
        mesh=plsc.VectorSubcoreMesh(core_axis_name="c", subcore_axis_name="s"),
        out_type=jax.ShapeDtypeStruct((S, D), jnp.float32),
        scratch_types=[
            pltpu.VMEM((SUB,), jnp.int32),
            pltpu.VMEM((SUB,), jnp.int32),
            pltpu.VMEM((SUB, D), jnp.float32),
            pltpu.VMEM((SUB, D), jnp.float32),
            pltpu.SemaphoreType.DMA,
            pltpu.SemaphoreType.DMA,
        ],
    )


def _combine_call(dp0_hbm, dp1_hbm, outs_hbm):
    return _combine_call_builder()(dp0_hbm, dp1_hbm, outs_hbm)


def kernel(x, Wr1, br1, Wg, We_in, We_out):
    bsz, length, d = x.shape
    x2d = x.reshape(S, D)
    dpos, gp16, counts = _router_call(x2d, Wr1, br1.reshape(1, -1), Wg)
    dposf = dpos.reshape(NPAIR)
    ci = counts.reshape(E)
    nt = (ci + TMC - 1) // TMC
    te = jnp.repeat(jnp.arange(E, dtype=jnp.int32), nt,
                    total_repeat_length=NT)
    xs, gs = _scatter_call(dposf, x2d, gp16)
    outs = _gmm_call(te, xs, gs, We_in, We_out)
    y2 = _combine_call(dposf[:S], dposf[S:], outs)
    loss = jnp.zeros((), dtype=jnp.float32)
    return y2.reshape(bsz, length, d), loss


# speedup vs baseline: 1.0522x; 1.0522x over previous
import functools

import jax
import jax.numpy as jnp
from jax import lax
from jax.experimental import pallas as pl
from jax.experimental.pallas import tpu as pltpu
from jax.experimental.pallas import tpu_sc as plsc

E = 8
K = 2
D = 768
H = 768
S = 2048
NPAIR = S * K
TMC = 256
NP = NPAIR + E * TMC
NT = NPAIR // TMC + E
NC = 2
NS = 16
NW = NC * NS
BP = NPAIR // NW
TW = S // NW
SUB = 32


def _router_kernel(x_ref, wr1_ref, br1_ref, wg_ref,
                   dpos_ref, gp_ref, cnt_ref):
    xb = x_ref[...]
    h = lax.dot_general(xb, wr1_ref[...], (((1,), (1,)), ((), ())),
                        preferred_element_type=jnp.float32)
    h = jnp.tanh(h + br1_ref[...])
    logits = lax.dot_general(h, wg_ref[...], (((1,), (1,)), ((), ())),
                             preferred_element_type=jnp.float32)
    m = jnp.max(logits, axis=1, keepdims=True)
    p = jnp.exp(logits - m)
    p = p / jnp.sum(p, axis=1, keepdims=True)
    e_iota = lax.broadcasted_iota(jnp.int32, p.shape, 1)
    m1 = jnp.max(p, axis=1, keepdims=True)
    i1 = jnp.min(jnp.where(p == m1, e_iota, E), axis=1, keepdims=True)
    p_rest = jnp.where(e_iota == i1, -jnp.inf, p)
    m2 = jnp.max(p_rest, axis=1, keepdims=True)
    i2 = jnp.min(jnp.where(p_rest == m2, e_iota, E), axis=1, keepdims=True)
    denom = m1 + m2 + 1e-6
    g1 = m1 / denom
    g2 = m2 / denom

    e8 = lax.broadcasted_iota(jnp.int32, (S, E), 1)
    oh1 = (e8 == i1).astype(jnp.float32)
    oh2 = (e8 == i2).astype(jnp.float32)
    oh = jnp.concatenate([oh1, oh2], axis=0)

    r_i = lax.broadcasted_iota(jnp.int32, (256, 256), 0)
    c_i = lax.broadcasted_iota(jnp.int32, (256, 256), 1)
    tri = (c_i < r_i).astype(jnp.float32)

    nblk = NPAIR // 256
    t_rows = []
    rank_blocks = []
    for b in range(nblk):
        ohb = oh[b * 256:(b + 1) * 256, :]
        rb = lax.dot_general(tri, ohb, (((1,), (0,)), ((), ())),
                             preferred_element_type=jnp.float32)
        rank_blocks.append(rb)
        t_rows.append(jnp.sum(ohb, axis=0, keepdims=True))
    t_mat = jnp.concatenate(t_rows, axis=0)

    r16 = lax.broadcasted_iota(jnp.int32, (nblk, nblk), 0)
    c16 = lax.broadcasted_iota(jnp.int32, (nblk, nblk), 1)
    tri16 = (r16 < c16).astype(jnp.float32)
    bo = lax.dot_general(tri16, t_mat, (((0,), (0,)), ((), ())),
                         preferred_element_type=jnp.float32)

    counts = jnp.sum(t_mat, axis=0, keepdims=True)
    ci = counts.astype(jnp.int32)
    pci = ((ci + (TMC - 1)) // TMC) * TMC
    pcf = pci.astype(jnp.float32)
    r8 = lax.broadcasted_iota(jnp.int32, (E, E), 0)
    c8 = lax.broadcasted_iota(jnp.int32, (E, E), 1)
    tri8 = (r8 < c8).astype(jnp.float32)
    eo = lax.dot_general(pcf, tri8, (((1,), (0,)), ((), ())),
                         preferred_element_type=jnp.float32)

    dpos_parts = []
    for b in range(nblk):
        ohb = oh[b * 256:(b + 1) * 256, :]
        base = rank_blocks[b] + bo[b:b + 1, :] + eo
        dpos_parts.append(jnp.sum(base * ohb, axis=1, keepdims=True))
    dpos = jnp.concatenate(dpos_parts, axis=0)
    dpos_ref[...] = dpos.astype(jnp.int32)

    gpair = jnp.concatenate([g1, g2], axis=0)
    gp_ref[...] = jnp.broadcast_to(gpair, (NPAIR, 128))
    cnt_ref[...] = ci


def _router_call(x2d, Wr1, br1_2d, Wg, interpret=False):
    return pl.pallas_call(
        _router_kernel,
        in_specs=[
            pl.BlockSpec((S, D), lambda: (0, 0)),
            pl.BlockSpec((D, D), lambda: (0, 0)),
            pl.BlockSpec((1, D), lambda: (0, 0)),
            pl.BlockSpec((E, D), lambda: (0, 0)),
        ],
        out_specs=[
            pl.BlockSpec((NPAIR, 1), lambda: (0, 0)),
            pl.BlockSpec((NPAIR, 128), lambda: (0, 0)),
            pl.BlockSpec((1, E), lambda: (0, 0)),
        ],
        out_shape=[
            jax.ShapeDtypeStruct((NPAIR, 1), jnp.int32),
            jax.ShapeDtypeStruct((NPAIR, 128), jnp.float32),
            jax.ShapeDtypeStruct((1, E), jnp.int32),
        ],
        compiler_params=pltpu.CompilerParams(
            vmem_limit_bytes=60 * 1024 * 1024,
        ),
        interpret=interpret,
    )(x2d, Wr1, br1_2d, Wg)


def _scatter_kernel(dpos_hbm, x_hbm, gp_hbm, xs_hbm, gs_hbm,
                    idx_v, rows_v, g_v, sem1, sem2):
    wid = lax.axis_index("s") * NC + lax.axis_index("c")
    pbase = wid * BP
    tokbase = lax.rem(pbase, S)
    pltpu.sync_copy(dpos_hbm.at[pl.ds(pbase, BP)], idx_v)
    pltpu.sync_copy(x_hbm.at[pl.ds(tokbase, BP)], rows_v)
    pltpu.sync_copy(gp_hbm.at[pl.ds(pbase, BP)], g_v)
    cp1 = pltpu.async_copy(rows_v, xs_hbm.at[idx_v], sem1)
    cp2 = pltpu.async_copy(g_v, gs_hbm.at[idx_v], sem2)
    cp1.wait()
    cp2.wait()


@functools.lru_cache(maxsize=1)
def _scatter_call_builder():
    return pl.kernel(
        _scatter_kernel,
        mesh=plsc.VectorSubcoreMesh(core_axis_name="c", subcore_axis_name="s"),
        out_type=[
            jax.ShapeDtypeStruct((NP, D), jnp.float32),
            jax.ShapeDtypeStruct((NP, 128), jnp.float32),
        ],
        scratch_types=[
            pltpu.VMEM((BP,), jnp.int32),
            pltpu.VMEM((BP, D), jnp.float32),
            pltpu.VMEM((BP, 128), jnp.float32),
            pltpu.SemaphoreType.DMA,
            pltpu.SemaphoreType.DMA,
        ],
    )


def _scatter_call(dpos_hbm, x_hbm, gp_hbm):
    return _scatter_call_builder()(dpos_hbm, x_hbm, gp_hbm)


def _gmm_kernel(te_ref, xs_ref, gs_ref, win_ref, wout_ref, out_ref):
    del te_ref
    hh = lax.dot_general(xs_ref[...], win_ref[0], (((1,), (0,)), ((), ())),
                         preferred_element_type=jnp.float32)
    hh = jnp.maximum(hh, 0.0)
    oo = lax.dot_general(hh, wout_ref[0], (((1,), (0,)), ((), ())),
                         preferred_element_type=jnp.float32)
    out_ref[...] = oo * gs_ref[:, 0:1]


def _gmm_call(te, xs, gs, We_in, We_out, interpret=False):
    return pl.pallas_call(
        _gmm_kernel,
        grid_spec=pltpu.PrefetchScalarGridSpec(
            num_scalar_prefetch=1,
            grid=(NT,),
            in_specs=[
                pl.BlockSpec((TMC, D), lambda t, te_ref: (t, 0)),
                pl.BlockSpec((TMC, 128), lambda t, te_ref: (t, 0)),
                pl.BlockSpec((1, D, H), lambda t, te_ref: (te_ref[t], 0, 0)),
                pl.BlockSpec((1, H, D), lambda t, te_ref: (te_ref[t], 0, 0)),
            ],
            out_specs=pl.BlockSpec((TMC, D), lambda t, te_ref: (t, 0)),
        ),
        out_shape=jax.ShapeDtypeStruct((NP, D), jnp.float32),
        compiler_params=pltpu.CompilerParams(
            dimension_semantics=("arbitrary",),
            vmem_limit_bytes=60 * 1024 * 1024,
        ),
        interpret=interpret,
    )(te, xs, gs, We_in, We_out)


def _combine_kernel(dp0_hbm, dp1_hbm, outs_hbm, y_hbm,
                    p0_v, p1_v, r0_v, r1_v, sem0, sem1):
    wid = lax.axis_index("s") * NC + lax.axis_index("c")
    for sub in range(TW // SUB):
        base = wid * TW + sub * SUB
        pltpu.sync_copy(dp0_hbm.at[pl.ds(base, SUB)], p0_v)
        pltpu.sync_copy(dp1_hbm.at[pl.ds(base, SUB)], p1_v)
        cp0 = pltpu.async_copy(outs_hbm.at[p0_v], r0_v, sem0)
        cp1 = pltpu.async_copy(outs_hbm.at[p1_v], r1_v, sem1)
        cp0.wait()
        cp1.wait()

        def _row(r, carry):
            for j in range(D // 16):
                sl = pl.ds(j * 16, 16)
                r0_v[r, sl] = r0_v[r, sl] + r1_v[r, sl]
            return carry

        lax.fori_loop(0, SUB, _row, 0)
        pltpu.sync_copy(r0_v, y_hbm.at[pl.ds(base, SUB)])


@functools.lru_cache(maxsize=1)
def _combine_call_builder():
    return pl.kernel(
        _combine_kernel,
        mesh=plsc.VectorSubcoreMesh(core_axis_name="c", subcore_axis_name="s"),
        out_type=jax.ShapeDtypeStruct((S, D), jnp.float32),
        scratch_types=[
            pltpu.VMEM((SUB,), jnp.int32),
            pltpu.VMEM((SUB,), jnp.int32),
            pltpu.VMEM((SUB, D), jnp.float32),
            pltpu.VMEM((SUB, D), jnp.float32),
            pltpu.SemaphoreType.DMA,
            pltpu.SemaphoreType.DMA,
        ],
    )


def _combine_call(dp0_hbm, dp1_hbm, outs_hbm):
    return _combine_call_builder()(dp0_hbm, dp1_hbm, outs_hbm)


def kernel(x, Wr1, br1, Wg, We_in, We_out):
    bsz, length, d = x.shape
    x2d = x.reshape(S, D)
    dpos, gp16, counts = _router_call(x2d, Wr1, br1.reshape(1, -1), Wg)
    dposf = dpos.reshape(NPAIR)
    ci = counts.reshape(E)
    nt = (ci + TMC - 1) // TMC
    te = jnp.repeat(jnp.arange(E, dtype=jnp.int32), nt,
                    total_repeat_length=NT)
    xs, gs = _scatter_call(dposf, x2d, gp16)
    outs = _gmm_call(te, xs, gs, We_in, We_out)
    y2 = _combine_call(dposf[:S], dposf[S:], outs)
    loss = jnp.zeros((), dtype=jnp.float32)
    return y2.reshape(bsz, length, d), loss
